# baseline (speedup 1.0000x reference)
import jax
import jax.numpy as jnp
from jax import lax
from jax.experimental import pallas as pl
from jax.experimental.pallas import tpu as pltpu
from jax.experimental.pallas import tpu_sc as plsc

N_ATOMS = 6400000
NUM_EMB = 104
NUM_SEG = 4096
TAB_REP = 16
TAB_PAD = 128 * TAB_REP

NUM_CORES = 2
NUM_SUBCORES = 16
NUM_WORKERS = NUM_CORES * NUM_SUBCORES
CHUNK = N_ATOMS // NUM_WORKERS
BLK = 20000
NUM_BLKS = CHUNK // BLK
SB = 160
SB_VECS = SB // 16
NUM_SBS = BLK // SB
NUM_GRPS = (NUM_SBS + 15) // 16
SEG_SLICE = NUM_SEG // NUM_SUBCORES


def _take16(x, idx):
    return x.at[idx].get(mode="promise_in_bounds")


def _sc_body(species_hbm, batch_hbm, table_hbm, out_hbm,
             table_v, sp0_v, b0_v, sp1_v, b1_v, acc_v, red_v, res_v, shared,
             sem_s0, sem_b0, sem_s1, sem_b1):
    cid = lax.axis_index("c")
    sid = lax.axis_index("s")
    wid = cid * NUM_SUBCORES + sid
    base = wid * CHUNK

    pltpu.sync_copy(table_hbm, table_v)

    zeros16 = jnp.zeros((16,), jnp.float32)

    def zero_body(q, _):
        acc_v[pl.ds(q * 16, 16)] = zeros16
        return _

    lax.fori_loop(0, NUM_SEG // 16, zero_body, None)

    iota = lax.iota(jnp.int32, 16)
    iota0 = jnp.zeros((16,), jnp.int32)

    def gather_shifts(sp):
        return plsc.load_gather(table_v, [(sp << 4) + iota])
    idxp1 = jnp.minimum(iota + 1, 15)
    is_lane15 = iota == 15

    bufs = [(sp0_v, b0_v), (sp1_v, b1_v)]
    sems = [(sem_s0, sem_b0), (sem_s1, sem_b1)]
    pending = {}

    def issue(blk):
        pb = blk % 2
        off = base + blk * BLK
        c1 = pltpu.async_copy(species_hbm.at[pl.ds(off, BLK)],
                              bufs[pb][0], sems[pb][0])
        c2 = pltpu.async_copy(batch_hbm.at[pl.ds(off, BLK)],
                              bufs[pb][1], sems[pb][1])
        pending[blk] = (c1, c2)

    issue(0)
    for blk in range(NUM_BLKS):
        if blk + 1 < NUM_BLKS:
            issue(blk + 1)
        c1, c2 = pending.pop(blk)
        c1.wait()
        c2.wait()
        sp_v, b_v = bufs[blk % 2]

        @plsc.parallel_loop(0, NUM_SBS, step=1)
        def sb_body(q):
            sb0 = q * SB
            bfirst = b_v[pl.ds(sb0, 16)]
            blast = b_v[pl.ds(sb0 + SB - 16, 16)]
            bf0 = _take16(bfirst, iota0)
            m15u = is_lane15 & (bf0 == blast)
            accs = [zeros16, zeros16, zeros16, zeros16]
            for t in range(SB_VECS):
                sp = sp_v[pl.ds(sb0 + t * 16, 16)]
                vals = gather_shifts(sp)
                accs[t % 4] = accs[t % 4] + vals
            total = plsc.cumsum(accs[0] + accs[1] + accs[2] + accs[3])
            plsc.addupdate_scatter(acc_v, [blast], total, mask=m15u)

        def slow_sb(sbq):
            for t in range(SB_VECS):
                off16 = sbq * SB + t * 16
                sp = sp_v[pl.ds(off16, 16)]
                b = b_v[pl.ds(off16, 16)]
                vals = gather_shifts(sp)
                c = plsc.cumsum(vals)
                bn = _take16(b, idxp1)
                neq = b != bn
                m1 = neq | is_lane15
                plsc.addupdate_scatter(acc_v, [b], c, mask=m1)
                plsc.addupdate_scatter(acc_v, [bn], 0.0 - c, mask=neq)

        def grp_body(g, _):
            gbase = g * 16
            sbi = jnp.minimum(gbase + iota, NUM_SBS - 1)
            valid = (gbase + iota) < NUM_SBS
            idxf = sbi * SB
            vf = plsc.load_gather(b_v, [idxf])
            vl = plsc.load_gather(b_v, [idxf + (SB - 1)])
            m0 = (vf != vl) & valid

            def w_cond(m):
                return jnp.any(m)

            def w_body(m):
                lane = plsc.all_reduce_ffs(m)
                lane0 = lane[0] if lane.ndim else lane
                slow_sb(gbase + lane0)
                return m & (iota != lane0)

            lax.while_loop(w_cond, w_body, m0)
            return _

        lax.fori_loop(0, NUM_GRPS, grp_body, None)

    pltpu.sync_copy(acc_v, shared.at[sid])
    plsc.subcore_barrier()

    col = sid * SEG_SLICE
    row_copies = [
        pltpu.async_copy(shared.at[r, pl.ds(col, SEG_SLICE)],
                         red_v.at[pl.ds(r * SEG_SLICE, SEG_SLICE)], sem_s0)
        for r in range(NUM_SUBCORES)
    ]
    for cpy in row_copies:
        cpy.wait()

    def red_body(q, _):
        v = jnp.zeros((16,), jnp.float32)
        for r in range(NUM_SUBCORES):
            v = v + red_v[pl.ds(r * SEG_SLICE + q * 16, 16)]
        res_v[pl.ds(q * 16, 16)] = v
        return _

    lax.fori_loop(0, SEG_SLICE // 16, red_body, None)

    pltpu.sync_copy(res_v, out_hbm.at[cid, pl.ds(col, SEG_SLICE)])


@jax.jit
def _sc_segsum(species, batch, table):
    mesh = plsc.VectorSubcoreMesh(core_axis_name="c", subcore_axis_name="s")
    return pl.kernel(
        _sc_body,
        out_type=jax.ShapeDtypeStruct((NUM_CORES, NUM_SEG), jnp.float32),
        mesh=mesh,
        compiler_params=pltpu.CompilerParams(needs_layout_passes=False),
        scratch_types=[
            pltpu.VMEM((TAB_PAD,), jnp.float32),
            pltpu.VMEM((BLK,), jnp.int32),
            pltpu.VMEM((BLK,), jnp.int32),
            pltpu.VMEM((BLK,), jnp.int32),
            pltpu.VMEM((BLK,), jnp.int32),
            pltpu.VMEM((NUM_SEG,), jnp.float32),
            pltpu.VMEM((NUM_SEG,), jnp.float32),
            pltpu.VMEM((SEG_SLICE,), jnp.float32),
            pltpu.VMEM_SHARED((NUM_SUBCORES, NUM_SEG), jnp.float32),
            pltpu.SemaphoreType.DMA,
            pltpu.SemaphoreType.DMA,
            pltpu.SemaphoreType.DMA,
            pltpu.SemaphoreType.DMA,
        ],
    )(species, batch, table)


def _add2_body(p_ref, o_ref):
    o_ref[...] = p_ref[0] + p_ref[1]


@jax.jit
def _add_partials(partials):
    p = partials.reshape(NUM_CORES, 32, 128)
    out = pl.pallas_call(
        _add2_body,
        out_shape=jax.ShapeDtypeStruct((32, 128), jnp.float32),
    )(p)
    return out.reshape(NUM_SEG)


def kernel(species, batch, constant_shifts):
    rep = jnp.repeat(constant_shifts[:, 0], TAB_REP)
    table = jnp.pad(rep, (0, TAB_PAD - NUM_EMB * TAB_REP))
    partials = _sc_segsum(species, batch, table)
    return _add_partials(partials)


# speedup vs baseline: 1.2377x; 1.2377x over previous
import jax
import jax.numpy as jnp
from jax import lax
from jax.experimental import pallas as pl
from jax.experimental.pallas import tpu as pltpu
from jax.experimental.pallas import tpu_sc as plsc

N_ATOMS = 6400000
NUM_EMB = 104
NUM_SEG = 4096
TAB_REP = 16
TAB_PAD = 128 * TAB_REP

NUM_CORES = 2
NUM_SUBCORES = 16
NUM_WORKERS = NUM_CORES * NUM_SUBCORES
CHUNK = N_ATOMS // NUM_WORKERS
BLK = 20000
NUM_BLKS = CHUNK // BLK
SB = 160
SB_VECS = SB // 16
NUM_SBS = BLK // SB
NUM_GRPS = (NUM_SBS + 15) // 16
SEG_SLICE = NUM_SEG // NUM_SUBCORES


def _take16(x, idx):
    return x.at[idx].get(mode="promise_in_bounds")


def _sc_body(species_hbm, batch_hbm, table_hbm, out_hbm,
             table_v, sp0_v, b0_v, sp1_v, b1_v, acc_v, red_v, res_v, shared,
             sem_s0, sem_b0, sem_s1, sem_b1):
    cid = lax.axis_index("c")
    sid = lax.axis_index("s")
    wid = cid * NUM_SUBCORES + sid
    base = wid * CHUNK

    bufs = [(sp0_v, b0_v), (sp1_v, b1_v)]
    sems = [(sem_s0, sem_b0), (sem_s1, sem_b1)]
    pending = {}

    def issue(blk):
        pb = blk % 2
        off = base + blk * BLK
        c1 = pltpu.async_copy(species_hbm.at[pl.ds(off, BLK)],
                              bufs[pb][0], sems[pb][0])
        c2 = pltpu.async_copy(batch_hbm.at[pl.ds(off, BLK)],
                              bufs[pb][1], sems[pb][1])
        pending[blk] = (c1, c2)

    issue(0)
    issue(1)

    pltpu.sync_copy(table_hbm, table_v)

    zeros16 = jnp.zeros((16,), jnp.float32)

    def zero_body(q, _):
        acc_v[pl.ds(q * 16, 16)] = zeros16
        return _

    lax.fori_loop(0, NUM_SEG // 16, zero_body, None)

    iota = lax.iota(jnp.int32, 16)
    iota0 = jnp.zeros((16,), jnp.int32)

    def gather_shifts(sp):
        return plsc.load_gather(table_v, [(sp << 4) + iota])
    idxp1 = jnp.minimum(iota + 1, 15)
    is_lane15 = iota == 15

    for blk in range(NUM_BLKS):
        if 2 <= blk + 1 < NUM_BLKS:
            issue(blk + 1)
        c1, c2 = pending.pop(blk)
        c1.wait()
        c2.wait()
        sp_v, b_v = bufs[blk % 2]

        @plsc.parallel_loop(0, NUM_SBS, step=1, unroll=2)
        def sb_body(q):
            sb0 = q * SB
            bfirst = b_v[pl.ds(sb0, 16)]
            blast = b_v[pl.ds(sb0 + SB - 16, 16)]
            bf0 = _take16(bfirst, iota0)
            m15u = is_lane15 & (bf0 == blast)
            accs = [zeros16, zeros16, zeros16, zeros16]
            for t in range(SB_VECS):
                sp = sp_v[pl.ds(sb0 + t * 16, 16)]
                vals = gather_shifts(sp)
                accs[t % 4] = accs[t % 4] + vals
            total = plsc.cumsum(accs[0] + accs[1] + accs[2] + accs[3])
            plsc.addupdate_scatter(acc_v, [blast], total, mask=m15u)

        def slow_sb(sbq):
            @plsc.parallel_loop(0, SB_VECS, step=1)
            def vec_body(t):
                off16 = sbq * SB + t * 16
                sp = sp_v[pl.ds(off16, 16)]
                b = b_v[pl.ds(off16, 16)]
                vals = gather_shifts(sp)
                c = plsc.cumsum(vals)
                bn = _take16(b, idxp1)
                neq = b != bn
                m1 = neq | is_lane15
                plsc.addupdate_scatter(acc_v, [b], c, mask=m1)
                plsc.addupdate_scatter(acc_v, [bn], 0.0 - c, mask=neq)

        def grp_body(g, _):
            gbase = g * 16
            sbi = jnp.minimum(gbase + iota, NUM_SBS - 1)
            valid = (gbase + iota) < NUM_SBS
            idxf = sbi * SB
            vf = plsc.load_gather(b_v, [idxf])
            vl = plsc.load_gather(b_v, [idxf + (SB - 1)])
            m0 = (vf != vl) & valid

            def w_cond(m):
                return jnp.any(m)

            def w_body(m):
                lane = plsc.all_reduce_ffs(m)
                lane0 = lane[0] if lane.ndim else lane
                slow_sb(gbase + lane0)
                return m & (iota != lane0)

            lax.while_loop(w_cond, w_body, m0)
            return _

        lax.fori_loop(0, NUM_GRPS, grp_body, None)

    pltpu.sync_copy(acc_v, shared.at[sid])
    plsc.subcore_barrier()

    col = sid * SEG_SLICE
    row_copies = [
        pltpu.async_copy(shared.at[r, pl.ds(col, SEG_SLICE)],
                         red_v.at[pl.ds(r * SEG_SLICE, SEG_SLICE)], sem_s0)
        for r in range(NUM_SUBCORES)
    ]
    for cpy in row_copies:
        cpy.wait()

    def red_body(q, _):
        v = jnp.zeros((16,), jnp.float32)
        for r in range(NUM_SUBCORES):
            v = v + red_v[pl.ds(r * SEG_SLICE + q * 16, 16)]
        res_v[pl.ds(q * 16, 16)] = v
        return _

    lax.fori_loop(0, SEG_SLICE // 16, red_body, None)

    pltpu.sync_copy(res_v, out_hbm.at[cid, pl.ds(col, SEG_SLICE)])


@jax.jit
def _sc_segsum(species, batch, table):
    mesh = plsc.VectorSubcoreMesh(core_axis_name="c", subcore_axis_name="s")
    return pl.kernel(
        _sc_body,
        out_type=jax.ShapeDtypeStruct((NUM_CORES, NUM_SEG), jnp.float32),
        mesh=mesh,
        compiler_params=pltpu.CompilerParams(needs_layout_passes=False),
        scratch_types=[
            pltpu.VMEM((TAB_PAD,), jnp.float32),
            pltpu.VMEM((BLK,), jnp.int32),
            pltpu.VMEM((BLK,), jnp.int32),
            pltpu.VMEM((BLK,), jnp.int32),
            pltpu.VMEM((BLK,), jnp.int32),
            pltpu.VMEM((NUM_SEG,), jnp.float32),
            pltpu.VMEM((NUM_SEG,), jnp.float32),
            pltpu.VMEM((SEG_SLICE,), jnp.float32),
            pltpu.VMEM_SHARED((NUM_SUBCORES, NUM_SEG), jnp.float32),
            pltpu.SemaphoreType.DMA,
            pltpu.SemaphoreType.DMA,
            pltpu.SemaphoreType.DMA,
            pltpu.SemaphoreType.DMA,
        ],
    )(species, batch, table)


def _add2_body(p_ref, o_ref):
    o_ref[...] = p_ref[0] + p_ref[1]


@jax.jit
def _add_partials(partials):
    p = partials.reshape(NUM_CORES, 32, 128)
    out = pl.pallas_call(
        _add2_body,
        out_shape=jax.ShapeDtypeStruct((32, 128), jnp.float32),
    )(p)
    return out.reshape(NUM_SEG)


def kernel(species, batch, constant_shifts):
    rep = jnp.repeat(constant_shifts[:, 0], TAB_REP)
    table = jnp.pad(rep, (0, TAB_PAD - NUM_EMB * TAB_REP))
    partials = _sc_segsum(species, batch, table)
    return _add_partials(partials)


# speedup vs baseline: 1.2634x; 1.0207x over previous
import jax
import jax.numpy as jnp
from jax import lax
from jax.experimental import pallas as pl
from jax.experimental.pallas import tpu as pltpu
from jax.experimental.pallas import tpu_sc as plsc

N_ATOMS = 6400000
NUM_EMB = 104
NUM_SEG = 4096
TAB_REP = 16
TAB_PAD = 128 * TAB_REP

NUM_CORES = 2
NUM_SUBCORES = 16
NUM_WORKERS = NUM_CORES * NUM_SUBCORES
CHUNK = N_ATOMS // NUM_WORKERS
BLK = 20000
NUM_BLKS = CHUNK // BLK
SB = 160
SB_VECS = SB // 16
NUM_SBS = BLK // SB
NUM_GRPS = (NUM_SBS + 15) // 16
SEG_SLICE = NUM_SEG // NUM_SUBCORES


def _take16(x, idx):
    return x.at[idx].get(mode="promise_in_bounds")


def _sc_body(species_hbm, batch_hbm, table_hbm, out_hbm,
             table_v, sp0_v, b0_v, sp1_v, b1_v, acc_v, red_v, res_v, shared,
             sem_s0, sem_b0, sem_s1, sem_b1):
    cid = lax.axis_index("c")
    sid = lax.axis_index("s")
    wid = cid * NUM_SUBCORES + sid
    base = wid * CHUNK

    bufs = [(sp0_v, b0_v), (sp1_v, b1_v)]
    sems = [(sem_s0, sem_b0), (sem_s1, sem_b1)]
    pending = {}

    def issue(blk):
        pb = blk % 2
        off = base + blk * BLK
        c1 = pltpu.async_copy(species_hbm.at[pl.ds(off, BLK)],
                              bufs[pb][0], sems[pb][0])
        c2 = pltpu.async_copy(batch_hbm.at[pl.ds(off, BLK)],
                              bufs[pb][1], sems[pb][1])
        pending[blk] = (c1, c2)

    issue(0)
    issue(1)

    pltpu.sync_copy(table_hbm, table_v)

    zeros16 = jnp.zeros((16,), jnp.float32)

    def zero_body(q, _):
        acc_v[pl.ds(q * 16, 16)] = zeros16
        return _

    lax.fori_loop(0, NUM_SEG // 16, zero_body, None)

    iota = lax.iota(jnp.int32, 16)
    iota0 = jnp.zeros((16,), jnp.int32)

    def gather_shifts(sp):
        return plsc.load_gather(table_v, [(sp << 4) + iota])
    idxp1 = jnp.minimum(iota + 1, 15)
    is_lane15 = iota == 15

    for blk in range(NUM_BLKS):
        if 2 <= blk + 1 < NUM_BLKS:
            issue(blk + 1)
        c1, c2 = pending.pop(blk)
        c1.wait()
        c2.wait()
        sp_v, b_v = bufs[blk % 2]

        @plsc.parallel_loop(0, NUM_SBS, step=1)
        def sb_body(q):
            sb0 = q * SB
            bfirst = b_v[pl.ds(sb0, 16)]
            blast = b_v[pl.ds(sb0 + SB - 16, 16)]
            bf0 = _take16(bfirst, iota0)
            m15u = is_lane15 & (bf0 == blast)
            accs = [zeros16, zeros16, zeros16, zeros16]
            for t in range(SB_VECS):
                sp = sp_v[pl.ds(sb0 + t * 16, 16)]
                vals = gather_shifts(sp)
                accs[t % 4] = accs[t % 4] + vals
            total = plsc.cumsum(accs[0] + accs[1] + accs[2] + accs[3])
            plsc.addupdate_scatter(acc_v, [blast], total, mask=m15u)

        def slow_sb(sbq):
            @plsc.parallel_loop(0, SB_VECS, step=1)
            def vec_body(t):
                off16 = sbq * SB + t * 16
                sp = sp_v[pl.ds(off16, 16)]
                b = b_v[pl.ds(off16, 16)]
                vals = gather_shifts(sp)
                c = plsc.cumsum(vals)
                bn = _take16(b, idxp1)
                neq = b != bn
                m1 = neq | is_lane15
                plsc.addupdate_scatter(acc_v, [b], c, mask=m1)
                plsc.addupdate_scatter(acc_v, [bn], 0.0 - c, mask=neq)

        def grp_body(g, _):
            gbase = g * 16
            sbi = jnp.minimum(gbase + iota, NUM_SBS - 1)
            valid = (gbase + iota) < NUM_SBS
            idxf = sbi * SB
            vf = plsc.load_gather(b_v, [idxf])
            vl = plsc.load_gather(b_v, [idxf + (SB - 1)])
            m0 = (vf != vl) & valid

            def w_cond(m):
                return jnp.any(m)

            def w_body(m):
                lane = plsc.all_reduce_ffs(m)
                lane0 = lane[0] if lane.ndim else lane
                slow_sb(gbase + lane0)
                return m & (iota != lane0)

            lax.while_loop(w_cond, w_body, m0)
            return _

        lax.fori_loop(0, NUM_GRPS, grp_body, None)

    pltpu.sync_copy(acc_v, shared.at[sid])
    plsc.subcore_barrier()

    col = sid * SEG_SLICE
    row_copies = [
        pltpu.async_copy(shared.at[r, pl.ds(col, SEG_SLICE)],
                         red_v.at[pl.ds(r * SEG_SLICE, SEG_SLICE)], sem_s0)
        for r in range(NUM_SUBCORES)
    ]
    for cpy in row_copies:
        cpy.wait()

    def red_body(q, _):
        v = jnp.zeros((16,), jnp.float32)
        for r in range(NUM_SUBCORES):
            v = v + red_v[pl.ds(r * SEG_SLICE + q * 16, 16)]
        res_v[pl.ds(q * 16, 16)] = v
        return _

    lax.fori_loop(0, SEG_SLICE // 16, red_body, None)

    pltpu.sync_copy(res_v, out_hbm.at[cid, pl.ds(col, SEG_SLICE)])


@jax.jit
def _sc_segsum(species, batch, table):
    mesh = plsc.VectorSubcoreMesh(core_axis_name="c", subcore_axis_name="s")
    return pl.kernel(
        _sc_body,
        out_type=jax.ShapeDtypeStruct((NUM_CORES, NUM_SEG), jnp.float32),
        mesh=mesh,
        compiler_params=pltpu.CompilerParams(needs_layout_passes=False),
        scratch_types=[
            pltpu.VMEM((TAB_PAD,), jnp.float32),
            pltpu.VMEM((BLK,), jnp.int32),
            pltpu.VMEM((BLK,), jnp.int32),
            pltpu.VMEM((BLK,), jnp.int32),
            pltpu.VMEM((BLK,), jnp.int32),
            pltpu.VMEM((NUM_SEG,), jnp.float32),
            pltpu.VMEM((NUM_SEG,), jnp.float32),
            pltpu.VMEM((SEG_SLICE,), jnp.float32),
            pltpu.VMEM_SHARED((NUM_SUBCORES, NUM_SEG), jnp.float32),
            pltpu.SemaphoreType.DMA,
            pltpu.SemaphoreType.DMA,
            pltpu.SemaphoreType.DMA,
            pltpu.SemaphoreType.DMA,
        ],
    )(species, batch, table)


def _add2_body(p_ref, o_ref):
    o_ref[...] = p_ref[0] + p_ref[1]


@jax.jit
def _add_partials(partials):
    p = partials.reshape(NUM_CORES, 32, 128)
    out = pl.pallas_call(
        _add2_body,
        out_shape=jax.ShapeDtypeStruct((32, 128), jnp.float32),
    )(p)
    return out.reshape(NUM_SEG)


def kernel(species, batch, constant_shifts):
    rep = jnp.repeat(constant_shifts[:, 0], TAB_REP)
    table = jnp.pad(rep, (0, TAB_PAD - NUM_EMB * TAB_REP))
    partials = _sc_segsum(species, batch, table)
    return _add_partials(partials)
